# baseline (speedup 1.0000x reference)
import functools

import jax
import jax.numpy as jnp
from jax import lax
from jax.experimental import pallas as pl
from jax.experimental.pallas import tpu as pltpu
from jax.experimental.pallas import tpu_sc as plsc

B, S, H = 1, 2048, 768
E, K, DFF = 8, 2, 1024
T = B * S
NP = T * K
BT = 256
NB = NP // BT + E
P = NB * BT
ROUTER_AUX_COEF = 0.001
ROUTER_Z_COEF = 0.001


def _excl_cumsum0(x):
    c = x
    s = 1
    n = x.shape[0]
    while s < n:
        c = c + jnp.concatenate(
            [jnp.zeros((s, x.shape[1]), x.dtype), c[:-s, :]], axis=0)
        s *= 2
    return c - x


def _incl_cumsum1(x):
    c = x
    s = 1
    n = x.shape[1]
    while s < n:
        c = c + jnp.concatenate(
            [jnp.zeros((x.shape[0], s), x.dtype), c[:, :-s]], axis=1)
        s *= 2
    return c


def _router_body(x_ref, w_ref, b_ref,
                 tp_ref, ti_ref, aux_ref, pf_ref, bm_ref, w0_ref, w1_ref):
    xb = x_ref[...].astype(jnp.bfloat16)
    wb = w_ref[...].astype(jnp.bfloat16)
    logits = lax.dot_general(
        xb, wb, (((1,), (0,)), ((), ())),
        preferred_element_type=jnp.float32) + b_ref[...][None, :]
    m = jnp.max(logits, axis=-1, keepdims=True)
    ex = jnp.exp(logits - m)
    z = jnp.sum(ex, axis=-1, keepdims=True)
    p = ex / z

    lane = lax.broadcasted_iota(jnp.int32, (T, E), 1)
    v1 = jnp.max(p, axis=-1, keepdims=True)
    i1 = jnp.min(jnp.where(p == v1, lane, E), axis=-1, keepdims=True)
    p_m = jnp.where(lane == i1, -jnp.inf, p)
    v2 = jnp.max(p_m, axis=-1, keepdims=True)
    i2 = jnp.min(jnp.where(p_m == v2, lane, E), axis=-1, keepdims=True)

    tp_ref[...] = jnp.concatenate([v1, v2], axis=1)
    ti_ref[...] = jnp.concatenate([i1, i2], axis=1)
    oh1 = (lane == i1).astype(jnp.int32)
    oh2 = (lane == i2).astype(jnp.int32)

    mask = (oh1 + oh2).astype(jnp.float32)
    fraction = jnp.mean(mask, axis=0, keepdims=True)
    mean_prob = jnp.mean(p, axis=0, keepdims=True)
    lbl = E * jnp.sum(fraction * mean_prob, axis=1, keepdims=True)
    zm = jnp.maximum(v1, v2)
    lse = zm + jnp.log(jnp.exp(v1 - zm) + jnp.exp(v2 - zm))
    zl = jnp.mean(lse * lse, axis=0, keepdims=True)
    aux_ref[...] = lbl * ROUTER_AUX_COEF + zl * ROUTER_Z_COEF

    c0 = _excl_cumsum0(oh1)
    c1 = _excl_cumsum0(oh2)
    cnt0 = jnp.sum(oh1, axis=0, keepdims=True)
    cnt_t = cnt0 + jnp.sum(oh2, axis=0, keepdims=True)
    nb = (cnt_t + (BT - 1)) // BT
    end_blk = _incl_cumsum1(nb)
    start_blk = end_blk - nb
    group_start = start_blk * BT
    pf_ref[0:T] = jnp.sum(
        jnp.where(lane == i1, group_start + c0, 0), axis=1, keepdims=True)
    pf_ref[T:2 * T] = jnp.sum(
        jnp.where(lane == i2, group_start + cnt0 + c1, 0),
        axis=1, keepdims=True)

    b_iota = lax.broadcasted_iota(jnp.int32, (NB, E), 0)
    bm = jnp.sum((end_blk <= b_iota).astype(jnp.int32), axis=1, keepdims=True)
    bm_ref[0:NB] = jnp.minimum(bm, E - 1)
    bm_ref[NB:NB + 1] = end_blk[:, E - 1:E]

    ones128 = jnp.ones((1, 128), jnp.float32)
    w0_ref[...] = v1 * ones128
    w1_ref[...] = v2 * ones128


def _grouped_ffn_body(bm_ref, nb_ref, x_ref, w1_ref, b1_ref, w2_ref, b2_ref,
                      o_ref):
    b = pl.program_id(0)
    f = pl.program_id(1)

    @pl.when(b < nb_ref[0])
    def _():
        xb = x_ref[...].astype(jnp.bfloat16)
        h = lax.dot_general(
            xb, w1_ref[0].astype(jnp.bfloat16), (((1,), (0,)), ((), ())),
            preferred_element_type=jnp.float32) + b1_ref[0]
        h = jax.nn.gelu(h)
        y = lax.dot_general(
            h.astype(jnp.bfloat16), w2_ref[0].astype(jnp.bfloat16),
            (((1,), (0,)), ((), ())),
            preferred_element_type=jnp.float32)

        @pl.when(f == 0)
        def _():
            o_ref[...] = y + b2_ref[0]

        @pl.when(f != 0)
        def _():
            o_ref[...] += y


def _make_dispatch():
    info = plsc.get_sparse_core_info()
    nw = info.num_cores * info.num_subcores
    cpw = NP // nw
    mesh = plsc.VectorSubcoreMesh(core_axis_name="c", subcore_axis_name="s")

    @functools.partial(
        pl.kernel, mesh=mesh,
        out_type=jax.ShapeDtypeStruct((P, H), jnp.float32),
        scratch_types=[
            pltpu.VMEM((cpw // 2,), jnp.int32),
            pltpu.VMEM((cpw // 2,), jnp.int32),
            pltpu.VMEM((cpw // 2,), jnp.int32),
            pltpu.VMEM((cpw // 2,), jnp.int32),
            pltpu.VMEM((cpw // 2, H), jnp.float32),
            pltpu.VMEM((cpw // 2, H), jnp.float32),
            pltpu.SemaphoreType.DMA,
            pltpu.SemaphoreType.DMA,
        ],
    )
    def dispatch(tok_hbm, pos_hbm, xs_hbm,
                 pos_a, pos_b, tix_a, tix_b, rows_a, rows_b, s0, s1):
        wid = lax.axis_index("s") * info.num_cores + lax.axis_index("c")
        half = cpw // 2
        base = wid * cpw
        tbase = lax.rem(base, T)
        pltpu.sync_copy(pos_hbm.at[pl.ds(base, half)], pos_a)
        pltpu.sync_copy(pos_hbm.at[pl.ds(base + half, half)], pos_b)
        for j in range(half // 16):
            tix_a[pl.ds(j * 16, 16)] = lax.iota(jnp.int32, 16) + (
                tbase + j * 16)
            tix_b[pl.ds(j * 16, 16)] = lax.iota(jnp.int32, 16) + (
                tbase + half + j * 16)
        ga = pltpu.async_copy(tok_hbm.at[tix_a], rows_a, s0)
        gb = pltpu.async_copy(tok_hbm.at[tix_b], rows_b, s1)
        ga.wait()
        sa = pltpu.async_copy(rows_a, xs_hbm.at[pos_a], s0)
        gb.wait()
        sb = pltpu.async_copy(rows_b, xs_hbm.at[pos_b], s1)
        sa.wait()
        sb.wait()

    return dispatch


def _wadd_body(y0_ref, y1_ref, tp_ref, o_ref):
    tp = tp_ref[...]
    o_ref[...] = y0_ref[...] * tp[:, :1] + y1_ref[...] * tp[:, 1:2]


def _make_combine():
    info = plsc.get_sparse_core_info()
    nw = info.num_cores * info.num_subcores
    tpw = T // nw
    mesh = plsc.VectorSubcoreMesh(core_axis_name="c", subcore_axis_name="s")

    @functools.partial(
        pl.kernel, mesh=mesh,
        out_type=jax.ShapeDtypeStruct((T, H), jnp.float32),
        scratch_types=[
            pltpu.VMEM((tpw,), jnp.int32),
            pltpu.VMEM((tpw,), jnp.int32),
            pltpu.VMEM((tpw, 128), jnp.float32),
            pltpu.VMEM((tpw, 128), jnp.float32),
            pltpu.VMEM((tpw, H), jnp.float32),
            pltpu.VMEM((tpw, H), jnp.float32),
            pltpu.SemaphoreType.DMA,
            pltpu.SemaphoreType.DMA,
        ],
    )
    def combine(y_hbm, pos_hbm, w0_hbm, w1_hbm, out_hbm,
                p0_v, p1_v, w0_v, w1_v, y0_v, y1_v, sem0, sem1):
        wid = lax.axis_index("s") * info.num_cores + lax.axis_index("c")
        tb = wid * tpw
        pltpu.sync_copy(pos_hbm.at[pl.ds(tb, tpw)], p0_v)
        pltpu.sync_copy(pos_hbm.at[pl.ds(T + tb, tpw)], p1_v)
        g0 = pltpu.async_copy(y_hbm.at[p0_v], y0_v, sem0)
        g1 = pltpu.async_copy(y_hbm.at[p1_v], y1_v, sem1)
        pltpu.sync_copy(w0_hbm.at[pl.ds(tb, tpw)], w0_v)
        pltpu.sync_copy(w1_hbm.at[pl.ds(tb, tpw)], w1_v)
        g0.wait()
        g1.wait()

        def row(i, carry):
            wr0 = w0_v[i, pl.ds(0, 16)]
            wr1 = w1_v[i, pl.ds(0, 16)]
            for j in range(H // 16):
                sl = pl.ds(j * 16, 16)
                y0_v[i, sl] = y0_v[i, sl] * wr0 + y1_v[i, sl] * wr1
            return carry

        lax.fori_loop(0, tpw, row, 0)
        pltpu.sync_copy(y0_v, out_hbm.at[pl.ds(tb, tpw)])

    return combine


@jax.jit
def kernel(hidden_states, router_w, router_b, w1, b1, w2, b2):
    tokens = hidden_states.reshape(T, H)

    top_probs, top_idx, aux, posflat2, blkmap2, w0r, w1r = pl.pallas_call(
        _router_body,
        out_shape=(
            jax.ShapeDtypeStruct((T, K), jnp.float32),
            jax.ShapeDtypeStruct((T, K), jnp.int32),
            jax.ShapeDtypeStruct((1, 1), jnp.float32),
            jax.ShapeDtypeStruct((NP, 1), jnp.int32),
            jax.ShapeDtypeStruct((NB + 1, 1), jnp.int32),
            jax.ShapeDtypeStruct((T, 128), jnp.float32),
            jax.ShapeDtypeStruct((T, 128), jnp.float32),
        ),
    )(tokens, router_w, router_b)

    posflat = posflat2.reshape(NP)

    x_sorted = _make_dispatch()(tokens, posflat)

    FC = DFF // 2
    grid_spec = pltpu.PrefetchScalarGridSpec(
        num_scalar_prefetch=2,
        grid=(NB, 2),
        in_specs=[
            pl.BlockSpec((BT, H), lambda b, f, m, n: (b, 0)),
            pl.BlockSpec((1, H, FC), lambda b, f, m, n: (m[b], 0, f)),
            pl.BlockSpec((1, 1, FC), lambda b, f, m, n: (m[b], 0, f)),
            pl.BlockSpec((1, FC, H), lambda b, f, m, n: (m[b], f, 0)),
            pl.BlockSpec((1, 1, H), lambda b, f, m, n: (m[b], 0, 0)),
        ],
        out_specs=pl.BlockSpec((BT, H), lambda b, f, m, n: (b, 0)),
    )
    y_sorted = pl.pallas_call(
        _grouped_ffn_body,
        grid_spec=grid_spec,
        out_shape=jax.ShapeDtypeStruct((P, H), jnp.float32),
    )(blkmap2[:NB].reshape(NB), blkmap2[NB:].reshape(1),
      x_sorted, w1, b1.reshape(E, 1, DFF), w2, b2.reshape(E, 1, H))

    out = _make_combine()(y_sorted, posflat, w0r, w1r)

    output = out.reshape(B, S, H)
    aux_loss = aux[0, 0]
    route_probs = top_probs.reshape(B, S, K)
    route_indices = top_idx.reshape(B, S, K)
    return (output, aux_loss, route_probs, route_indices)


# speedup vs baseline: 1.2356x; 1.2356x over previous
import functools

import jax
import jax.numpy as jnp
from jax import lax
from jax.experimental import pallas as pl
from jax.experimental.pallas import tpu as pltpu
from jax.experimental.pallas import tpu_sc as plsc

B, S, H = 1, 2048, 768
E, K, DFF = 8, 2, 1024
T = B * S
NP = T * K
BT = 256
NB = NP // BT + E
P = NB * BT
ROUTER_AUX_COEF = 0.001
ROUTER_Z_COEF = 0.001


def _excl_cumsum0(x):
    c = x
    s = 1
    n = x.shape[0]
    while s < n:
        c = c + jnp.concatenate(
            [jnp.zeros((s, x.shape[1]), x.dtype), c[:-s, :]], axis=0)
        s *= 2
    return c - x


def _incl_cumsum1(x):
    c = x
    s = 1
    n = x.shape[1]
    while s < n:
        c = c + jnp.concatenate(
            [jnp.zeros((x.shape[0], s), x.dtype), c[:, :-s]], axis=1)
        s *= 2
    return c


def _router_body(x_ref, w_ref, b_ref,
                 tp_ref, ti_ref, aux_ref, pf_ref, bm_ref, w0_ref, w1_ref):
    xb = x_ref[...].astype(jnp.bfloat16)
    wb = w_ref[...].astype(jnp.bfloat16)
    logits = lax.dot_general(
        xb, wb, (((1,), (0,)), ((), ())),
        preferred_element_type=jnp.float32) + b_ref[...][None, :]
    m = jnp.max(logits, axis=-1, keepdims=True)
    ex = jnp.exp(logits - m)
    z = jnp.sum(ex, axis=-1, keepdims=True)
    p = ex / z

    lane = lax.broadcasted_iota(jnp.int32, (T, E), 1)
    v1 = jnp.max(p, axis=-1, keepdims=True)
    i1 = jnp.min(jnp.where(p == v1, lane, E), axis=-1, keepdims=True)
    p_m = jnp.where(lane == i1, -jnp.inf, p)
    v2 = jnp.max(p_m, axis=-1, keepdims=True)
    i2 = jnp.min(jnp.where(p_m == v2, lane, E), axis=-1, keepdims=True)

    tp_ref[...] = jnp.concatenate([v1, v2], axis=1)
    ti_ref[...] = jnp.concatenate([i1, i2], axis=1)
    oh1 = (lane == i1).astype(jnp.int32)
    oh2 = (lane == i2).astype(jnp.int32)

    mask = (oh1 + oh2).astype(jnp.float32)
    fraction = jnp.mean(mask, axis=0, keepdims=True)
    mean_prob = jnp.mean(p, axis=0, keepdims=True)
    lbl = E * jnp.sum(fraction * mean_prob, axis=1, keepdims=True)
    zm = jnp.maximum(v1, v2)
    lse = zm + jnp.log(jnp.exp(v1 - zm) + jnp.exp(v2 - zm))
    zl = jnp.mean(lse * lse, axis=0, keepdims=True)
    aux_ref[...] = lbl * ROUTER_AUX_COEF + zl * ROUTER_Z_COEF

    c0 = _excl_cumsum0(oh1)
    c1 = _excl_cumsum0(oh2)
    cnt0 = jnp.sum(oh1, axis=0, keepdims=True)
    cnt_t = cnt0 + jnp.sum(oh2, axis=0, keepdims=True)
    nb = (cnt_t + (BT - 1)) // BT
    end_blk = _incl_cumsum1(nb)
    start_blk = end_blk - nb
    group_start = start_blk * BT
    pf_ref[0:T] = jnp.sum(
        jnp.where(lane == i1, group_start + c0, 0), axis=1, keepdims=True)
    pf_ref[T:2 * T] = jnp.sum(
        jnp.where(lane == i2, group_start + cnt0 + c1, 0),
        axis=1, keepdims=True)

    b_iota = lax.broadcasted_iota(jnp.int32, (NB, E), 0)
    bm = jnp.sum((end_blk <= b_iota).astype(jnp.int32), axis=1, keepdims=True)
    bm_ref[0:NB] = jnp.minimum(bm, E - 1)
    bm_ref[NB:NB + 1] = end_blk[:, E - 1:E]

    ones128 = jnp.ones((1, 128), jnp.float32)
    w0_ref[...] = v1 * ones128
    w1_ref[...] = v2 * ones128


def _grouped_ffn_body(bm_ref, nb_ref, x_ref, w1_ref, b1_ref, w2_ref, b2_ref,
                      o_ref):
    b = pl.program_id(0)

    @pl.when(b < nb_ref[0])
    def _():
        xb = x_ref[...].astype(jnp.bfloat16)
        h = lax.dot_general(
            xb, w1_ref[0].astype(jnp.bfloat16), (((1,), (0,)), ((), ())),
            preferred_element_type=jnp.float32) + b1_ref[0]
        h = jax.nn.gelu(h)
        y = lax.dot_general(
            h.astype(jnp.bfloat16), w2_ref[0].astype(jnp.bfloat16),
            (((1,), (0,)), ((), ())),
            preferred_element_type=jnp.float32) + b2_ref[0]
        o_ref[...] = y


def _make_dispatch():
    info = plsc.get_sparse_core_info()
    nw = info.num_cores * info.num_subcores
    cpw = NP // nw
    mesh = plsc.VectorSubcoreMesh(core_axis_name="c", subcore_axis_name="s")

    @functools.partial(
        pl.kernel, mesh=mesh,
        out_type=jax.ShapeDtypeStruct((P, H), jnp.float32),
        scratch_types=[
            pltpu.VMEM((cpw // 2,), jnp.int32),
            pltpu.VMEM((cpw // 2,), jnp.int32),
            pltpu.VMEM((cpw // 2,), jnp.int32),
            pltpu.VMEM((cpw // 2,), jnp.int32),
            pltpu.VMEM((cpw // 2, H), jnp.float32),
            pltpu.VMEM((cpw // 2, H), jnp.float32),
            pltpu.SemaphoreType.DMA,
            pltpu.SemaphoreType.DMA,
        ],
    )
    def dispatch(tok_hbm, pos_hbm, xs_hbm,
                 pos_a, pos_b, tix_a, tix_b, rows_a, rows_b, s0, s1):
        wid = lax.axis_index("s") * info.num_cores + lax.axis_index("c")
        half = cpw // 2
        base = wid * cpw
        tbase = lax.rem(base, T)
        pltpu.sync_copy(pos_hbm.at[pl.ds(base, half)], pos_a)
        pltpu.sync_copy(pos_hbm.at[pl.ds(base + half, half)], pos_b)
        for j in range(half // 16):
            tix_a[pl.ds(j * 16, 16)] = lax.iota(jnp.int32, 16) + (
                tbase + j * 16)
            tix_b[pl.ds(j * 16, 16)] = lax.iota(jnp.int32, 16) + (
                tbase + half + j * 16)
        ga = pltpu.async_copy(tok_hbm.at[tix_a], rows_a, s0)
        gb = pltpu.async_copy(tok_hbm.at[tix_b], rows_b, s1)
        ga.wait()
        sa = pltpu.async_copy(rows_a, xs_hbm.at[pos_a], s0)
        gb.wait()
        sb = pltpu.async_copy(rows_b, xs_hbm.at[pos_b], s1)
        sa.wait()
        sb.wait()

    return dispatch


def _wadd_body(y0_ref, y1_ref, tp_ref, o_ref):
    tp = tp_ref[...]
    o_ref[...] = y0_ref[...] * tp[:, :1] + y1_ref[...] * tp[:, 1:2]


def _make_combine():
    info = plsc.get_sparse_core_info()
    nw = info.num_cores * info.num_subcores
    tpw = T // nw
    mesh = plsc.VectorSubcoreMesh(core_axis_name="c", subcore_axis_name="s")

    @functools.partial(
        pl.kernel, mesh=mesh,
        out_type=jax.ShapeDtypeStruct((T, H), jnp.float32),
        scratch_types=[
            pltpu.VMEM((tpw,), jnp.int32),
            pltpu.VMEM((tpw,), jnp.int32),
            pltpu.VMEM((tpw, 128), jnp.float32),
            pltpu.VMEM((tpw, 128), jnp.float32),
            pltpu.VMEM((tpw, H), jnp.float32),
            pltpu.VMEM((tpw, H), jnp.float32),
            pltpu.SemaphoreType.DMA,
            pltpu.SemaphoreType.DMA,
        ],
    )
    def combine(y_hbm, pos_hbm, w0_hbm, w1_hbm, out_hbm,
                p0_v, p1_v, w0_v, w1_v, y0_v, y1_v, sem0, sem1):
        wid = lax.axis_index("s") * info.num_cores + lax.axis_index("c")
        tb = wid * tpw
        pltpu.sync_copy(pos_hbm.at[pl.ds(tb, tpw)], p0_v)
        pltpu.sync_copy(pos_hbm.at[pl.ds(T + tb, tpw)], p1_v)
        g0 = pltpu.async_copy(y_hbm.at[p0_v], y0_v, sem0)
        g1 = pltpu.async_copy(y_hbm.at[p1_v], y1_v, sem1)
        pltpu.sync_copy(w0_hbm.at[pl.ds(tb, tpw)], w0_v)
        pltpu.sync_copy(w1_hbm.at[pl.ds(tb, tpw)], w1_v)
        g0.wait()
        g1.wait()

        def row(i, carry):
            wr0 = w0_v[i, pl.ds(0, 16)]
            wr1 = w1_v[i, pl.ds(0, 16)]
            for j in range(H // 16):
                sl = pl.ds(j * 16, 16)
                y0_v[i, sl] = y0_v[i, sl] * wr0 + y1_v[i, sl] * wr1
            return carry

        lax.fori_loop(0, tpw, row, 0)
        pltpu.sync_copy(y0_v, out_hbm.at[pl.ds(tb, tpw)])

    return combine


@jax.jit
def kernel(hidden_states, router_w, router_b, w1, b1, w2, b2):
    tokens = hidden_states.reshape(T, H)

    top_probs, top_idx, aux, posflat2, blkmap2, w0r, w1r = pl.pallas_call(
        _router_body,
        out_shape=(
            jax.ShapeDtypeStruct((T, K), jnp.float32),
            jax.ShapeDtypeStruct((T, K), jnp.int32),
            jax.ShapeDtypeStruct((1, 1), jnp.float32),
            jax.ShapeDtypeStruct((NP, 1), jnp.int32),
            jax.ShapeDtypeStruct((NB + 1, 1), jnp.int32),
            jax.ShapeDtypeStruct((T, 128), jnp.float32),
            jax.ShapeDtypeStruct((T, 128), jnp.float32),
        ),
    )(tokens, router_w, router_b)

    posflat = posflat2.reshape(NP)

    x_sorted = _make_dispatch()(tokens, posflat)

    grid_spec = pltpu.PrefetchScalarGridSpec(
        num_scalar_prefetch=2,
        grid=(NB,),
        in_specs=[
            pl.BlockSpec((BT, H), lambda b, m, n: (b, 0)),
            pl.BlockSpec((1, H, DFF), lambda b, m, n: (m[b], 0, 0)),
            pl.BlockSpec((1, 1, DFF), lambda b, m, n: (m[b], 0, 0)),
            pl.BlockSpec((1, DFF, H), lambda b, m, n: (m[b], 0, 0)),
            pl.BlockSpec((1, 1, H), lambda b, m, n: (m[b], 0, 0)),
        ],
        out_specs=pl.BlockSpec((BT, H), lambda b, m, n: (b, 0)),
    )
    y_sorted = pl.pallas_call(
        _grouped_ffn_body,
        grid_spec=grid_spec,
        out_shape=jax.ShapeDtypeStruct((P, H), jnp.float32),
    )(blkmap2[:NB].reshape(NB), blkmap2[NB:].reshape(1),
      x_sorted, w1, b1.reshape(E, 1, DFF), w2, b2.reshape(E, 1, H))

    out = _make_combine()(y_sorted, posflat, w0r, w1r)

    output = out.reshape(B, S, H)
    aux_loss = aux[0, 0]
    route_probs = top_probs.reshape(B, S, K)
    route_indices = top_idx.reshape(B, S, K)
    return (output, aux_loss, route_probs, route_indices)


# speedup vs baseline: 1.3185x; 1.0671x over previous
import functools

import jax
import jax.numpy as jnp
from jax import lax
from jax.experimental import pallas as pl
from jax.experimental.pallas import tpu as pltpu
from jax.experimental.pallas import tpu_sc as plsc

B, S, H = 1, 2048, 768
E, K, DFF = 8, 2, 1024
T = B * S
NP = T * K
BT = 512
NB = NP // BT + E
P = NB * BT
ROUTER_AUX_COEF = 0.001
ROUTER_Z_COEF = 0.001


def _excl_cumsum0(x):
    c = x
    s = 1
    n = x.shape[0]
    while s < n:
        c = c + jnp.concatenate(
            [jnp.zeros((s, x.shape[1]), x.dtype), c[:-s, :]], axis=0)
        s *= 2
    return c - x


def _incl_cumsum1(x):
    c = x
    s = 1
    n = x.shape[1]
    while s < n:
        c = c + jnp.concatenate(
            [jnp.zeros((x.shape[0], s), x.dtype), c[:, :-s]], axis=1)
        s *= 2
    return c


def _router_body(x_ref, w_ref, b_ref,
                 tp_ref, ti_ref, aux_ref, pf_ref, bm_ref, w0_ref, w1_ref):
    xb = x_ref[...].astype(jnp.bfloat16)
    wb = w_ref[...].astype(jnp.bfloat16)
    logits = lax.dot_general(
        xb, wb, (((1,), (0,)), ((), ())),
        preferred_element_type=jnp.float32) + b_ref[...][None, :]
    m = jnp.max(logits, axis=-1, keepdims=True)
    ex = jnp.exp(logits - m)
    z = jnp.sum(ex, axis=-1, keepdims=True)
    p = ex / z

    lane = lax.broadcasted_iota(jnp.int32, (T, E), 1)
    v1 = jnp.max(p, axis=-1, keepdims=True)
    i1 = jnp.min(jnp.where(p == v1, lane, E), axis=-1, keepdims=True)
    p_m = jnp.where(lane == i1, -jnp.inf, p)
    v2 = jnp.max(p_m, axis=-1, keepdims=True)
    i2 = jnp.min(jnp.where(p_m == v2, lane, E), axis=-1, keepdims=True)

    tp_ref[...] = jnp.concatenate([v1, v2], axis=1)
    ti_ref[...] = jnp.concatenate([i1, i2], axis=1)
    oh1 = (lane == i1).astype(jnp.int32)
    oh2 = (lane == i2).astype(jnp.int32)

    mask = (oh1 + oh2).astype(jnp.float32)
    fraction = jnp.mean(mask, axis=0, keepdims=True)
    mean_prob = jnp.mean(p, axis=0, keepdims=True)
    lbl = E * jnp.sum(fraction * mean_prob, axis=1, keepdims=True)
    zm = jnp.maximum(v1, v2)
    lse = zm + jnp.log(jnp.exp(v1 - zm) + jnp.exp(v2 - zm))
    zl = jnp.mean(lse * lse, axis=0, keepdims=True)
    aux_ref[...] = lbl * ROUTER_AUX_COEF + zl * ROUTER_Z_COEF

    c0 = _excl_cumsum0(oh1)
    c1 = _excl_cumsum0(oh2)
    cnt0 = jnp.sum(oh1, axis=0, keepdims=True)
    cnt_t = cnt0 + jnp.sum(oh2, axis=0, keepdims=True)
    nb = (cnt_t + (BT - 1)) // BT
    end_blk = _incl_cumsum1(nb)
    start_blk = end_blk - nb
    group_start = start_blk * BT
    pf_ref[0:T] = jnp.sum(
        jnp.where(lane == i1, group_start + c0, 0), axis=1, keepdims=True)
    pf_ref[T:2 * T] = jnp.sum(
        jnp.where(lane == i2, group_start + cnt0 + c1, 0),
        axis=1, keepdims=True)

    b_iota = lax.broadcasted_iota(jnp.int32, (NB, E), 0)
    bm = jnp.sum((end_blk <= b_iota).astype(jnp.int32), axis=1, keepdims=True)
    bm_ref[0:NB] = jnp.minimum(bm, E - 1)
    bm_ref[NB:NB + 1] = end_blk[:, E - 1:E]

    ones128 = jnp.ones((1, 128), jnp.float32)
    w0_ref[...] = v1 * ones128
    w1_ref[...] = v2 * ones128


def _grouped_ffn_body(bm_ref, nb_ref, x_ref, w1_ref, b1_ref, w2_ref, b2_ref,
                      o_ref):
    b = pl.program_id(0)

    @pl.when(b < nb_ref[0])
    def _():
        xb = x_ref[...].astype(jnp.bfloat16)
        h = lax.dot_general(
            xb, w1_ref[0].astype(jnp.bfloat16), (((1,), (0,)), ((), ())),
            preferred_element_type=jnp.float32) + b1_ref[0]
        h = jax.nn.gelu(h)
        y = lax.dot_general(
            h.astype(jnp.bfloat16), w2_ref[0].astype(jnp.bfloat16),
            (((1,), (0,)), ((), ())),
            preferred_element_type=jnp.float32) + b2_ref[0]
        o_ref[...] = y


def _make_dispatch():
    info = plsc.get_sparse_core_info()
    nw = info.num_cores * info.num_subcores
    cpw = NP // nw
    mesh = plsc.VectorSubcoreMesh(core_axis_name="c", subcore_axis_name="s")

    @functools.partial(
        pl.kernel, mesh=mesh,
        out_type=jax.ShapeDtypeStruct((P, H), jnp.float32),
        scratch_types=[
            pltpu.VMEM((cpw // 2,), jnp.int32),
            pltpu.VMEM((cpw // 2,), jnp.int32),
            pltpu.VMEM((cpw // 2,), jnp.int32),
            pltpu.VMEM((cpw // 2,), jnp.int32),
            pltpu.VMEM((cpw // 2, H), jnp.float32),
            pltpu.VMEM((cpw // 2, H), jnp.float32),
            pltpu.SemaphoreType.DMA,
            pltpu.SemaphoreType.DMA,
        ],
    )
    def dispatch(tok_hbm, pos_hbm, xs_hbm,
                 pos_a, pos_b, tix_a, tix_b, rows_a, rows_b, s0, s1):
        wid = lax.axis_index("s") * info.num_cores + lax.axis_index("c")
        half = cpw // 2
        base = wid * cpw
        tbase = lax.rem(base, T)
        pltpu.sync_copy(pos_hbm.at[pl.ds(base, half)], pos_a)
        pltpu.sync_copy(pos_hbm.at[pl.ds(base + half, half)], pos_b)
        for j in range(half // 16):
            tix_a[pl.ds(j * 16, 16)] = lax.iota(jnp.int32, 16) + (
                tbase + j * 16)
            tix_b[pl.ds(j * 16, 16)] = lax.iota(jnp.int32, 16) + (
                tbase + half + j * 16)
        ga = pltpu.async_copy(tok_hbm.at[tix_a], rows_a, s0)
        gb = pltpu.async_copy(tok_hbm.at[tix_b], rows_b, s1)
        ga.wait()
        sa = pltpu.async_copy(rows_a, xs_hbm.at[pos_a], s0)
        gb.wait()
        sb = pltpu.async_copy(rows_b, xs_hbm.at[pos_b], s1)
        sa.wait()
        sb.wait()

    return dispatch


def _wadd_body(y0_ref, y1_ref, tp_ref, o_ref):
    tp = tp_ref[...]
    o_ref[...] = y0_ref[...] * tp[:, :1] + y1_ref[...] * tp[:, 1:2]


def _make_combine():
    info = plsc.get_sparse_core_info()
    nw = info.num_cores * info.num_subcores
    tpw = T // nw
    mesh = plsc.VectorSubcoreMesh(core_axis_name="c", subcore_axis_name="s")

    @functools.partial(
        pl.kernel, mesh=mesh,
        out_type=jax.ShapeDtypeStruct((T, H), jnp.float32),
        scratch_types=[
            pltpu.VMEM((tpw,), jnp.int32),
            pltpu.VMEM((tpw,), jnp.int32),
            pltpu.VMEM((tpw, 128), jnp.float32),
            pltpu.VMEM((tpw, 128), jnp.float32),
            pltpu.VMEM((tpw, H), jnp.float32),
            pltpu.VMEM((tpw, H), jnp.float32),
            pltpu.SemaphoreType.DMA,
            pltpu.SemaphoreType.DMA,
        ],
    )
    def combine(y_hbm, pos_hbm, w0_hbm, w1_hbm, out_hbm,
                p0_v, p1_v, w0_v, w1_v, y0_v, y1_v, sem0, sem1):
        wid = lax.axis_index("s") * info.num_cores + lax.axis_index("c")
        tb = wid * tpw
        pltpu.sync_copy(pos_hbm.at[pl.ds(tb, tpw)], p0_v)
        pltpu.sync_copy(pos_hbm.at[pl.ds(T + tb, tpw)], p1_v)
        g0 = pltpu.async_copy(y_hbm.at[p0_v], y0_v, sem0)
        g1 = pltpu.async_copy(y_hbm.at[p1_v], y1_v, sem1)
        pltpu.sync_copy(w0_hbm.at[pl.ds(tb, tpw)], w0_v)
        pltpu.sync_copy(w1_hbm.at[pl.ds(tb, tpw)], w1_v)
        g0.wait()
        g1.wait()

        def row(i, carry):
            wr0 = w0_v[i, pl.ds(0, 16)]
            wr1 = w1_v[i, pl.ds(0, 16)]
            for j in range(H // 16):
                sl = pl.ds(j * 16, 16)
                y0_v[i, sl] = y0_v[i, sl] * wr0 + y1_v[i, sl] * wr1
            return carry

        lax.fori_loop(0, tpw, row, 0)
        pltpu.sync_copy(y0_v, out_hbm.at[pl.ds(tb, tpw)])

    return combine


@jax.jit
def kernel(hidden_states, router_w, router_b, w1, b1, w2, b2):
    tokens = hidden_states.reshape(T, H)

    top_probs, top_idx, aux, posflat2, blkmap2, w0r, w1r = pl.pallas_call(
        _router_body,
        out_shape=(
            jax.ShapeDtypeStruct((T, K), jnp.float32),
            jax.ShapeDtypeStruct((T, K), jnp.int32),
            jax.ShapeDtypeStruct((1, 1), jnp.float32),
            jax.ShapeDtypeStruct((NP, 1), jnp.int32),
            jax.ShapeDtypeStruct((NB + 1, 1), jnp.int32),
            jax.ShapeDtypeStruct((T, 128), jnp.float32),
            jax.ShapeDtypeStruct((T, 128), jnp.float32),
        ),
    )(tokens, router_w, router_b)

    posflat = posflat2.reshape(NP)

    x_sorted = _make_dispatch()(tokens, posflat)

    grid_spec = pltpu.PrefetchScalarGridSpec(
        num_scalar_prefetch=2,
        grid=(NB,),
        in_specs=[
            pl.BlockSpec((BT, H), lambda b, m, n: (b, 0)),
            pl.BlockSpec((1, H, DFF), lambda b, m, n: (m[b], 0, 0)),
            pl.BlockSpec((1, 1, DFF), lambda b, m, n: (m[b], 0, 0)),
            pl.BlockSpec((1, DFF, H), lambda b, m, n: (m[b], 0, 0)),
            pl.BlockSpec((1, 1, H), lambda b, m, n: (m[b], 0, 0)),
        ],
        out_specs=pl.BlockSpec((BT, H), lambda b, m, n: (b, 0)),
    )
    y_sorted = pl.pallas_call(
        _grouped_ffn_body,
        grid_spec=grid_spec,
        out_shape=jax.ShapeDtypeStruct((P, H), jnp.float32),
    )(blkmap2[:NB].reshape(NB), blkmap2[NB:].reshape(1),
      x_sorted, w1, b1.reshape(E, 1, DFF), w2, b2.reshape(E, 1, H))

    out = _make_combine()(y_sorted, posflat, w0r, w1r)

    output = out.reshape(B, S, H)
    aux_loss = aux[0, 0]
    route_probs = top_probs.reshape(B, S, K)
    route_indices = top_idx.reshape(B, S, K)
    return (output, aux_loss, route_probs, route_indices)


# speedup vs baseline: 1.3191x; 1.0004x over previous
import functools

import jax
import jax.numpy as jnp
from jax import lax
from jax.experimental import pallas as pl
from jax.experimental.pallas import tpu as pltpu
from jax.experimental.pallas import tpu_sc as plsc

B, S, H = 1, 2048, 768
E, K, DFF = 8, 2, 1024
T = B * S
NP = T * K
BT = 512
NB = NP // BT + E
P = NB * BT
ROUTER_AUX_COEF = 0.001
ROUTER_Z_COEF = 0.001


def _excl_cumsum0(x):
    c = x
    s = 1
    n = x.shape[0]
    while s < n:
        c = c + jnp.concatenate(
            [jnp.zeros((s, x.shape[1]), x.dtype), c[:-s, :]], axis=0)
        s *= 2
    return c - x


def _incl_cumsum1(x):
    c = x
    s = 1
    n = x.shape[1]
    while s < n:
        c = c + jnp.concatenate(
            [jnp.zeros((x.shape[0], s), x.dtype), c[:, :-s]], axis=1)
        s *= 2
    return c


def _router_body(x_ref, w_ref, b_ref,
                 tp_ref, ti_ref, aux_ref, pf_ref, bm_ref, w0_ref, w1_ref):
    xb = x_ref[...].astype(jnp.bfloat16)
    wb = w_ref[...].astype(jnp.bfloat16)
    logits = lax.dot_general(
        xb, wb, (((1,), (0,)), ((), ())),
        preferred_element_type=jnp.float32) + b_ref[...][None, :]
    m = jnp.max(logits, axis=-1, keepdims=True)
    ex = jnp.exp(logits - m)
    z = jnp.sum(ex, axis=-1, keepdims=True)
    p = ex / z

    lane = lax.broadcasted_iota(jnp.int32, (T, E), 1)
    v1 = jnp.max(p, axis=-1, keepdims=True)
    i1 = jnp.min(jnp.where(p == v1, lane, E), axis=-1, keepdims=True)
    p_m = jnp.where(lane == i1, -jnp.inf, p)
    v2 = jnp.max(p_m, axis=-1, keepdims=True)
    i2 = jnp.min(jnp.where(p_m == v2, lane, E), axis=-1, keepdims=True)

    tp_ref[...] = jnp.concatenate([v1, v2], axis=1)
    ti_ref[...] = jnp.concatenate([i1, i2], axis=1)
    oh1 = (lane == i1).astype(jnp.int32)
    oh2 = (lane == i2).astype(jnp.int32)

    mask = (oh1 + oh2).astype(jnp.float32)
    fraction = jnp.mean(mask, axis=0, keepdims=True)
    mean_prob = jnp.mean(p, axis=0, keepdims=True)
    lbl = E * jnp.sum(fraction * mean_prob, axis=1, keepdims=True)
    zm = jnp.maximum(v1, v2)
    lse = zm + jnp.log(jnp.exp(v1 - zm) + jnp.exp(v2 - zm))
    zl = jnp.mean(lse * lse, axis=0, keepdims=True)
    aux_ref[...] = lbl * ROUTER_AUX_COEF + zl * ROUTER_Z_COEF

    cc = _excl_cumsum0(jnp.concatenate([oh1, oh2], axis=1))
    c0 = cc[:, :E]
    c1 = cc[:, E:]
    cnt0 = jnp.sum(oh1, axis=0, keepdims=True)
    cnt_t = cnt0 + jnp.sum(oh2, axis=0, keepdims=True)
    nb = (cnt_t + (BT - 1)) // BT
    end_blk = _incl_cumsum1(nb)
    start_blk = end_blk - nb
    group_start = start_blk * BT
    pf_ref[0:T] = jnp.sum(
        jnp.where(lane == i1, group_start + c0, 0), axis=1, keepdims=True)
    pf_ref[T:2 * T] = jnp.sum(
        jnp.where(lane == i2, group_start + cnt0 + c1, 0),
        axis=1, keepdims=True)

    b_iota = lax.broadcasted_iota(jnp.int32, (NB, E), 0)
    bm = jnp.sum((end_blk <= b_iota).astype(jnp.int32), axis=1, keepdims=True)
    bm_ref[0:NB] = jnp.minimum(bm, E - 1)
    bm_ref[NB:NB + 1] = end_blk[:, E - 1:E]

    ones128 = jnp.ones((1, 128), jnp.float32)
    w0_ref[...] = v1 * ones128
    w1_ref[...] = v2 * ones128


def _grouped_ffn_body(bm_ref, nb_ref, x_ref, w1_ref, b1_ref, w2_ref, b2_ref,
                      o_ref):
    b = pl.program_id(0)

    @pl.when(b < nb_ref[0])
    def _():
        xb = x_ref[...].astype(jnp.bfloat16)
        h = lax.dot_general(
            xb, w1_ref[0].astype(jnp.bfloat16), (((1,), (0,)), ((), ())),
            preferred_element_type=jnp.float32) + b1_ref[0]
        h = jax.nn.gelu(h)
        y = lax.dot_general(
            h.astype(jnp.bfloat16), w2_ref[0].astype(jnp.bfloat16),
            (((1,), (0,)), ((), ())),
            preferred_element_type=jnp.float32) + b2_ref[0]
        o_ref[...] = y


def _make_dispatch():
    info = plsc.get_sparse_core_info()
    nw = info.num_cores * info.num_subcores
    cpw = NP // nw
    mesh = plsc.VectorSubcoreMesh(core_axis_name="c", subcore_axis_name="s")

    @functools.partial(
        pl.kernel, mesh=mesh,
        out_type=jax.ShapeDtypeStruct((P, H), jnp.float32),
        scratch_types=[
            pltpu.VMEM((cpw // 2,), jnp.int32),
            pltpu.VMEM((cpw // 2,), jnp.int32),
            pltpu.VMEM((cpw // 2,), jnp.int32),
            pltpu.VMEM((cpw // 2,), jnp.int32),
            pltpu.VMEM((cpw // 2, H), jnp.float32),
            pltpu.VMEM((cpw // 2, H), jnp.float32),
            pltpu.SemaphoreType.DMA,
            pltpu.SemaphoreType.DMA,
        ],
    )
    def dispatch(tok_hbm, pos_hbm, xs_hbm,
                 pos_a, pos_b, tix_a, tix_b, rows_a, rows_b, s0, s1):
        wid = lax.axis_index("s") * info.num_cores + lax.axis_index("c")
        half = cpw // 2
        base = wid * cpw
        tbase = lax.rem(base, T)
        pltpu.sync_copy(pos_hbm.at[pl.ds(base, half)], pos_a)
        pltpu.sync_copy(pos_hbm.at[pl.ds(base + half, half)], pos_b)
        for j in range(half // 16):
            tix_a[pl.ds(j * 16, 16)] = lax.iota(jnp.int32, 16) + (
                tbase + j * 16)
            tix_b[pl.ds(j * 16, 16)] = lax.iota(jnp.int32, 16) + (
                tbase + half + j * 16)
        ga = pltpu.async_copy(tok_hbm.at[tix_a], rows_a, s0)
        gb = pltpu.async_copy(tok_hbm.at[tix_b], rows_b, s1)
        ga.wait()
        sa = pltpu.async_copy(rows_a, xs_hbm.at[pos_a], s0)
        gb.wait()
        sb = pltpu.async_copy(rows_b, xs_hbm.at[pos_b], s1)
        sa.wait()
        sb.wait()

    return dispatch


def _wadd_body(y0_ref, y1_ref, tp_ref, o_ref):
    tp = tp_ref[...]
    o_ref[...] = y0_ref[...] * tp[:, :1] + y1_ref[...] * tp[:, 1:2]


def _make_combine():
    info = plsc.get_sparse_core_info()
    nw = info.num_cores * info.num_subcores
    tpw = T // nw
    mesh = plsc.VectorSubcoreMesh(core_axis_name="c", subcore_axis_name="s")

    @functools.partial(
        pl.kernel, mesh=mesh,
        out_type=jax.ShapeDtypeStruct((T, H), jnp.float32),
        scratch_types=[
            pltpu.VMEM((tpw,), jnp.int32),
            pltpu.VMEM((tpw,), jnp.int32),
            pltpu.VMEM((tpw, 128), jnp.float32),
            pltpu.VMEM((tpw, 128), jnp.float32),
            pltpu.VMEM((tpw, H), jnp.float32),
            pltpu.VMEM((tpw, H), jnp.float32),
            pltpu.SemaphoreType.DMA,
            pltpu.SemaphoreType.DMA,
        ],
    )
    def combine(y_hbm, pos_hbm, w0_hbm, w1_hbm, out_hbm,
                p0_v, p1_v, w0_v, w1_v, y0_v, y1_v, sem0, sem1):
        wid = lax.axis_index("s") * info.num_cores + lax.axis_index("c")
        tb = wid * tpw
        pltpu.sync_copy(pos_hbm.at[pl.ds(tb, tpw)], p0_v)
        pltpu.sync_copy(pos_hbm.at[pl.ds(T + tb, tpw)], p1_v)
        g0 = pltpu.async_copy(y_hbm.at[p0_v], y0_v, sem0)
        g1 = pltpu.async_copy(y_hbm.at[p1_v], y1_v, sem1)
        pltpu.sync_copy(w0_hbm.at[pl.ds(tb, tpw)], w0_v)
        pltpu.sync_copy(w1_hbm.at[pl.ds(tb, tpw)], w1_v)
        g0.wait()
        g1.wait()

        def row(i, carry):
            wr0 = w0_v[i, pl.ds(0, 16)]
            wr1 = w1_v[i, pl.ds(0, 16)]
            for j in range(H // 16):
                sl = pl.ds(j * 16, 16)
                y0_v[i, sl] = y0_v[i, sl] * wr0 + y1_v[i, sl] * wr1
            return carry

        lax.fori_loop(0, tpw, row, 0)
        pltpu.sync_copy(y0_v, out_hbm.at[pl.ds(tb, tpw)])

    return combine


@jax.jit
def kernel(hidden_states, router_w, router_b, w1, b1, w2, b2):
    tokens = hidden_states.reshape(T, H)

    top_probs, top_idx, aux, posflat2, blkmap2, w0r, w1r = pl.pallas_call(
        _router_body,
        out_shape=(
            jax.ShapeDtypeStruct((T, K), jnp.float32),
            jax.ShapeDtypeStruct((T, K), jnp.int32),
            jax.ShapeDtypeStruct((1, 1), jnp.float32),
            jax.ShapeDtypeStruct((NP, 1), jnp.int32),
            jax.ShapeDtypeStruct((NB + 1, 1), jnp.int32),
            jax.ShapeDtypeStruct((T, 128), jnp.float32),
            jax.ShapeDtypeStruct((T, 128), jnp.float32),
        ),
    )(tokens, router_w, router_b)

    posflat = posflat2.reshape(NP)

    x_sorted = _make_dispatch()(tokens, posflat)

    grid_spec = pltpu.PrefetchScalarGridSpec(
        num_scalar_prefetch=2,
        grid=(NB,),
        in_specs=[
            pl.BlockSpec((BT, H), lambda b, m, n: (b, 0)),
            pl.BlockSpec((1, H, DFF), lambda b, m, n: (m[b], 0, 0)),
            pl.BlockSpec((1, 1, DFF), lambda b, m, n: (m[b], 0, 0)),
            pl.BlockSpec((1, DFF, H), lambda b, m, n: (m[b], 0, 0)),
            pl.BlockSpec((1, 1, H), lambda b, m, n: (m[b], 0, 0)),
        ],
        out_specs=pl.BlockSpec((BT, H), lambda b, m, n: (b, 0)),
    )
    y_sorted = pl.pallas_call(
        _grouped_ffn_body,
        grid_spec=grid_spec,
        out_shape=jax.ShapeDtypeStruct((P, H), jnp.float32),
    )(blkmap2[:NB].reshape(NB), blkmap2[NB:].reshape(1),
      x_sorted, w1, b1.reshape(E, 1, DFF), w2, b2.reshape(E, 1, H))

    out = _make_combine()(y_sorted, posflat, w0r, w1r)

    output = out.reshape(B, S, H)
    aux_loss = aux[0, 0]
    route_probs = top_probs.reshape(B, S, K)
    route_indices = top_idx.reshape(B, S, K)
    return (output, aux_loss, route_probs, route_indices)
